# baseline (speedup 1.0000x reference)
import functools

import jax
import jax.numpy as jnp
from jax import lax
from jax.experimental import pallas as pl
from jax.experimental.pallas import tpu as pltpu
from jax.experimental.pallas import tpu_sc as plsc

_WIN_BASES = (560, 350, 120)
_NWIN = 3
_WLEN = 20
_NC = 2
_NS = 16
_NW = _NC * _NS
_LANES = 16
_HCHUNK = 8


def _sc_body(x_hbm, out_hbm, rows_v, res_v, sem):
    B = 2
    C = x_hbm.shape[0] // (B * 224)
    H = 224
    W = x_hbm.shape[1]
    nh = H // _HCHUNK
    ntask = B * _NWIN * nh
    kmax = (ntask + _NW - 1) // _NW
    nk = W // _LANES
    inv = jnp.float32(1.0 / _WLEN)

    c = lax.axis_index("c")
    s = lax.axis_index("s")
    wid = s * _NC + c

    for k in range(kmax):
        t = k * _NW + wid

        @pl.when(t < ntask)
        def _():
            p = t // nh
            h = t % nh
            b = p // _NWIN
            wi = p % _NWIN
            base = jnp.where(wi == 0, _WIN_BASES[0],
                             jnp.where(wi == 1, _WIN_BASES[1], _WIN_BASES[2]))
            row0 = (b * C + base) * H + h * _HCHUNK

            cps = []
            for j in range(_WLEN):
                cps.append(
                    pltpu.async_copy(
                        x_hbm.at[pl.ds(row0 + j * H, _HCHUNK), :],
                        rows_v.at[j],
                        sem,
                    )
                )
            for cp in cps:
                cp.wait()

            def chunk(i, carry):
                r = i // nk
                kk = i % nk
                acc = rows_v[0, r, pl.ds(kk * _LANES, _LANES)]
                for j in range(1, _WLEN):
                    acc = acc + rows_v[j, r, pl.ds(kk * _LANES, _LANES)]
                res_v[r, pl.ds(kk * _LANES, _LANES)] = acc * inv
                return carry

            lax.fori_loop(0, _HCHUNK * nk, chunk, 0)

            orow0 = (b * _NWIN + wi) * H + h * _HCHUNK
            pltpu.sync_copy(res_v, out_hbm.at[pl.ds(orow0, _HCHUNK), :])


def kernel(x):
    B, C, H, W = x.shape
    x2d = x.reshape(B * C * H, W)

    run = functools.partial(
        pl.kernel,
        out_type=jax.ShapeDtypeStruct((B * _NWIN * H, W), jnp.float32),
        mesh=plsc.VectorSubcoreMesh(core_axis_name="c", subcore_axis_name="s"),
        scratch_types=[
            pltpu.VMEM((_WLEN, _HCHUNK, W), jnp.float32),
            pltpu.VMEM((_HCHUNK, W), jnp.float32),
            pltpu.SemaphoreType.DMA,
        ],
    )(_sc_body)

    out = run(x2d)
    return out.reshape(B, _NWIN, H, W)


# speedup vs baseline: 4.0046x; 4.0046x over previous
import functools

import jax
import jax.numpy as jnp
from jax import lax
from jax.experimental import pallas as pl
from jax.experimental.pallas import tpu as pltpu
from jax.experimental.pallas import tpu_sc as plsc

_WIN_BASES = (560, 350, 120)
_NWIN = 3
_WLEN = 20
_NC = 2
_NS = 16
_NW = _NC * _NS
_LANES = 16
_HCHUNK = 8


def _sc_body(x_hbm, out_hbm, rows_v, res_v, sem):
    B, C, H, W = x_hbm.shape
    nh = H // _HCHUNK
    ntask = B * _NWIN * nh
    kmax = (ntask + _NW - 1) // _NW
    nk = W // _LANES
    inv = jnp.float32(1.0 / _WLEN)

    c = lax.axis_index("c")
    s = lax.axis_index("s")
    wid = s * _NC + c

    for k in range(kmax):
        t = k * _NW + wid

        @pl.when(t < ntask)
        def _():
            p = t // nh
            h = t % nh
            b = p // _NWIN
            wi = p % _NWIN
            base = jnp.where(wi == 0, _WIN_BASES[0],
                             jnp.where(wi == 1, _WIN_BASES[1], _WIN_BASES[2]))
            h0 = h * _HCHUNK

            cps = []
            for j in range(_WLEN):
                cps.append(
                    pltpu.async_copy(
                        x_hbm.at[b, base + j, pl.ds(h0, _HCHUNK), :],
                        rows_v.at[j],
                        sem,
                    )
                )
            for cp in cps:
                cp.wait()

            def chunk(i, carry):
                r = i // nk
                kk = i % nk
                acc = rows_v[0, r, pl.ds(kk * _LANES, _LANES)]
                for j in range(1, _WLEN):
                    acc = acc + rows_v[j, r, pl.ds(kk * _LANES, _LANES)]
                res_v[r, pl.ds(kk * _LANES, _LANES)] = acc * inv
                return carry

            lax.fori_loop(0, _HCHUNK * nk, chunk, 0)

            pltpu.sync_copy(res_v, out_hbm.at[b, wi, pl.ds(h0, _HCHUNK), :])


def kernel(x):
    B, C, H, W = x.shape

    run = functools.partial(
        pl.kernel,
        out_type=jax.ShapeDtypeStruct((B, _NWIN, H, W), jnp.float32),
        mesh=plsc.VectorSubcoreMesh(core_axis_name="c", subcore_axis_name="s"),
        scratch_types=[
            pltpu.VMEM((_WLEN, _HCHUNK, W), jnp.float32),
            pltpu.VMEM((_HCHUNK, W), jnp.float32),
            pltpu.SemaphoreType.DMA,
        ],
    )(_sc_body)

    return run(x)


# speedup vs baseline: 13.5709x; 3.3888x over previous
import functools

import jax
import jax.numpy as jnp
from jax import lax
from jax.experimental import pallas as pl
from jax.experimental.pallas import tpu as pltpu
from jax.experimental.pallas import tpu_sc as plsc

_WIN_BASES = (560, 350, 120)
_NWIN = 3
_WLEN = 20
_NC = 2
_NS = 16
_NW = _NC * _NS
_LANES = 16
_WCH = 112
_HOCT = 8

_SLABS = ((0, 256, 0), (256, 128, 256), (512, 128, 384))
_DST_BASE = (432, 350, 120)


def _sc_body(y_hbm, out_hbm, buf_v, res_v, sem0, sem1, osem):
    B, H, W, C = y_hbm.shape
    noct = H // _HOCT
    ntask = B * noct
    kmax = (ntask + _NW - 1) // _NW
    nwc = W // _WCH
    nsub = _HOCT * nwc
    ng = _WCH // _LANES
    inv = jnp.float32(1.0 / _WLEN)
    sems = (sem0, sem1)

    c = lax.axis_index("c")
    s = lax.axis_index("s")
    wid = s * _NC + c

    def coords(m):
        k = m // nsub
        i = m % nsub
        t = k * _NW + wid
        b = t // noct
        h0 = pl.multiple_of((t % noct) * _HOCT, _HOCT)
        hof = i // nwc
        wc = i % nwc
        return t, k, b, h0, hof, wc

    def copies(m):
        t, k, b, h0, hof, wc = coords(m)
        w0 = pl.multiple_of(wc * _WCH, _WCH)
        for (so, width, do) in _SLABS:
            yield (
                y_hbm.at[b, h0 + hof, pl.ds(w0, _WCH), pl.ds(so, width)],
                buf_v.at[m % 2, :, pl.ds(do, width)],
                sems,
                m % 2,
            ), t

    def issue(m):
        for (src, dst, ss, par), t in copies(m):

            @pl.when(t < ntask)
            def _():
                @pl.when(par == 0)
                def _():
                    pltpu.async_copy(src, dst, ss[0])

                @pl.when(par == 1)
                def _():
                    pltpu.async_copy(src, dst, ss[1])

    def drain_in(m):
        for (src, dst, ss, par), t in copies(m):

            @pl.when(t < ntask)
            def _():
                @pl.when(par == 0)
                def _():
                    pltpu.make_async_copy(src, dst, ss[0]).wait()

                @pl.when(par == 1)
                def _():
                    pltpu.make_async_copy(src, dst, ss[1]).wait()

    iota = lax.iota(jnp.int32, _LANES)
    nm = kmax * nsub

    issue(0)

    def step(m, carry):
        t, k, b, h0, hof, wc = coords(m)
        issue(m + 1)
        drain_in(m)

        @pl.when(t < ntask)
        def _():
            def group(g, carry2):
                idx_w = g * _LANES + iota
                for wi in range(_NWIN):
                    acc = plsc.load_gather(
                        buf_v.at[m % 2],
                        [idx_w, jnp.full((_LANES,), _DST_BASE[wi], jnp.int32)],
                    )
                    for j in range(1, _WLEN):
                        acc = acc + plsc.load_gather(
                            buf_v.at[m % 2],
                            [idx_w,
                             jnp.full((_LANES,), _DST_BASE[wi] + j, jnp.int32)],
                        )
                    res_v[k, wi, hof,
                          pl.ds(wc * _WCH + g * _LANES, _LANES)] = acc * inv
                return carry2

            lax.fori_loop(0, ng, group, 0)

            @pl.when((m + 1) % nsub == 0)
            def _():
                for wi in range(_NWIN):
                    pltpu.async_copy(
                        res_v.at[k, wi],
                        out_hbm.at[b, wi, pl.ds(h0, _HOCT), :],
                        osem,
                    )

        return carry

    lax.fori_loop(0, nm, step, 0)

    for kk in range(kmax):
        t, k, b, h0, _, _ = coords(kk * nsub)

        @pl.when(t < ntask)
        def _():
            for wi in range(_NWIN):
                pltpu.make_async_copy(
                    res_v.at[k, wi],
                    out_hbm.at[b, wi, pl.ds(h0, _HOCT), :],
                    osem,
                ).wait()


def kernel(x):
    B, C, H, W = x.shape
    y = jnp.transpose(x, (0, 2, 3, 1))

    run = functools.partial(
        pl.kernel,
        out_type=jax.ShapeDtypeStruct((B, _NWIN, H, W), jnp.float32),
        mesh=plsc.VectorSubcoreMesh(core_axis_name="c", subcore_axis_name="s"),
        compiler_params=pltpu.CompilerParams(needs_layout_passes=False),
        scratch_types=[
            pltpu.VMEM((2, _WCH, 512), jnp.float32),
            pltpu.VMEM((2, _NWIN, _HOCT, W), jnp.float32),
            pltpu.SemaphoreType.DMA,
            pltpu.SemaphoreType.DMA,
            pltpu.SemaphoreType.DMA,
        ],
    )(_sc_body)

    return run(y)
